# baseline (speedup 1.0000x reference)
import dataclasses

import jax
import jax.numpy as jnp
from jax.experimental import pallas as pl
from jax.experimental.pallas import tpu as pltpu
from jax.experimental.pallas import tpu_sc as plsc

WINDOW = 256


def kernel(tgt, emb_weight):
    batch, hist = tgt.shape
    n = batch * hist
    idx = tgt.reshape(1, n).astype(jnp.int32)
    vocab, dim = emb_weight.shape

    mesh = plsc.VectorSubcoreMesh(core_axis_name="core",
                                  subcore_axis_name="subcore")

    cp = pltpu.CompilerParams()
    if "needs_layout_passes" in pltpu.CompilerParams.__dataclass_fields__:
        cp = dataclasses.replace(cp, needs_layout_passes=False)

    @pl.kernel(out_type=jax.ShapeDtypeStruct((n, dim), emb_weight.dtype),
               mesh=mesh,
               compiler_params=cp,
               scratch_types=[pltpu.VMEM((vocab, dim), jnp.float32)])
    def gather_kernel(table_hbm, idx_hbm, out_hbm, table_vmem):
        pltpu.sync_copy(table_hbm, table_vmem)

        def body(idx_vmem, out_vmem):
            @pl.loop(0, WINDOW, step=16)
            def _(r0):
                idx_vec = idx_vmem[0, pl.ds(r0, 16)]
                row_vec = r0 + jax.lax.iota(jnp.int32, 16)

                @pl.loop(0, dim, step=1)
                def _(c):
                    c_vec = jnp.full((16,), c, jnp.int32)
                    vals = plsc.load_gather(table_vmem, [idx_vec, c_vec])
                    plsc.store_scatter(out_vmem, [row_vec, c_vec], vals)

        pltpu.emit_pipeline(
            body,
            grid=(n // WINDOW,),
            in_specs=[pl.BlockSpec((1, WINDOW), index_map=lambda i: (0, i))],
            out_specs=[pl.BlockSpec((WINDOW, dim), index_map=lambda i: (i, 0))],
            core_axis_name=("core", "subcore"),
            dimension_semantics=(pltpu.PARALLEL,),
        )(idx_hbm, out_hbm)

    out = gather_kernel(emb_weight, idx)
    return out.reshape(batch, hist, dim)


# speedup vs baseline: 1.0019x; 1.0019x over previous
import dataclasses

import jax
import jax.numpy as jnp
from jax.experimental import pallas as pl
from jax.experimental.pallas import tpu as pltpu
from jax.experimental.pallas import tpu_sc as plsc

WINDOW = 256


def kernel(tgt, emb_weight):
    batch, hist = tgt.shape
    n = batch * hist
    idx = tgt.reshape(1, n).astype(jnp.int32)
    vocab, dim = emb_weight.shape

    mesh = plsc.VectorSubcoreMesh(core_axis_name="core",
                                  subcore_axis_name="subcore")

    cp = pltpu.CompilerParams()
    if "needs_layout_passes" in pltpu.CompilerParams.__dataclass_fields__:
        cp = dataclasses.replace(cp, needs_layout_passes=False)

    @pl.kernel(out_type=jax.ShapeDtypeStruct((n, dim), emb_weight.dtype),
               mesh=mesh,
               compiler_params=cp,
               scratch_types=[pltpu.VMEM((vocab, dim), jnp.float32)])
    def gather_kernel(table_hbm, idx_hbm, out_hbm, table_vmem):
        pltpu.sync_copy(table_hbm, table_vmem)

        def body(idx_vmem, out_vmem):
            @pl.loop(0, WINDOW, step=16)
            def _(r0):
                idx_vec = idx_vmem[0, pl.ds(r0, 16)]
                row_vec = r0 + jax.lax.iota(jnp.int32, 16)

                @pl.loop(0, dim, step=16)
                def _(c):
                    for k in range(16):
                        c_vec = jnp.full((16,), c + k, jnp.int32)
                        vals = plsc.load_gather(table_vmem, [idx_vec, c_vec])
                        plsc.store_scatter(out_vmem, [row_vec, c_vec], vals)

        pltpu.emit_pipeline(
            body,
            grid=(n // WINDOW,),
            in_specs=[pl.BlockSpec((1, WINDOW), index_map=lambda i: (0, i))],
            out_specs=[pl.BlockSpec((WINDOW, dim), index_map=lambda i: (i, 0))],
            core_axis_name=("core", "subcore"),
            dimension_semantics=(pltpu.PARALLEL,),
        )(idx_hbm, out_hbm)

    out = gather_kernel(emb_weight, idx)
    return out.reshape(batch, hist, dim)


# speedup vs baseline: 10.1069x; 10.0874x over previous
import jax
import jax.numpy as jnp
from jax.experimental import pallas as pl
from jax.experimental.pallas import tpu as pltpu
from jax.experimental.pallas import tpu_sc as plsc

WINDOW = 256
REP = 32


def kernel(tgt, emb_weight):
    batch, hist = tgt.shape
    n = batch * hist
    vocab, dim = emb_weight.shape
    table_rep = jnp.tile(emb_weight, (REP, 1))
    spread = (jnp.arange(n, dtype=jnp.int32) % REP) * vocab
    idx = (tgt.reshape(-1).astype(jnp.int32) + spread).reshape(1, n)

    mesh = plsc.VectorSubcoreMesh(core_axis_name="core",
                                  subcore_axis_name="subcore")

    @pl.kernel(out_type=jax.ShapeDtypeStruct((n, dim), emb_weight.dtype),
               mesh=mesh,
               scratch_types=[pltpu.SemaphoreType.DMA])
    def gather_kernel(table_hbm, idx_hbm, out_hbm, sem):
        def body(idx_vmem, out_vmem):
            pltpu.async_copy(table_hbm.at[idx_vmem.at[0]], out_vmem, sem).wait()

        pltpu.emit_pipeline(
            body,
            grid=(n // WINDOW,),
            in_specs=[pl.BlockSpec((1, WINDOW), index_map=lambda i: (0, i))],
            out_specs=[pl.BlockSpec((WINDOW, dim), index_map=lambda i: (i, 0))],
            core_axis_name=("core", "subcore"),
            dimension_semantics=(pltpu.PARALLEL,),
        )(idx_hbm, out_hbm)

    out = gather_kernel(table_rep, idx)
    return out.reshape(batch, hist, dim)


# speedup vs baseline: 11.5989x; 1.1476x over previous
import jax
import jax.numpy as jnp
from jax.experimental import pallas as pl
from jax.experimental.pallas import tpu as pltpu
from jax.experimental.pallas import tpu_sc as plsc

WINDOW = 256
REP = 128


def kernel(tgt, emb_weight):
    batch, hist = tgt.shape
    n = batch * hist
    vocab, dim = emb_weight.shape
    table_rep = jnp.tile(emb_weight, (REP, 1))
    spread = (jnp.arange(n, dtype=jnp.int32) % REP) * vocab
    idx = (tgt.reshape(-1).astype(jnp.int32) + spread).reshape(1, n)

    mesh = plsc.VectorSubcoreMesh(core_axis_name="core",
                                  subcore_axis_name="subcore")

    @pl.kernel(out_type=jax.ShapeDtypeStruct((n, dim), emb_weight.dtype),
               mesh=mesh,
               scratch_types=[pltpu.SemaphoreType.DMA])
    def gather_kernel(table_hbm, idx_hbm, out_hbm, sem):
        def body(idx_vmem, out_vmem):
            pltpu.async_copy(table_hbm.at[idx_vmem.at[0]], out_vmem, sem).wait()

        pltpu.emit_pipeline(
            body,
            grid=(n // WINDOW,),
            in_specs=[pl.BlockSpec((1, WINDOW), index_map=lambda i: (0, i))],
            out_specs=[pl.BlockSpec((WINDOW, dim), index_map=lambda i: (i, 0))],
            core_axis_name=("core", "subcore"),
            dimension_semantics=(pltpu.PARALLEL,),
        )(idx_hbm, out_hbm)

    out = gather_kernel(table_rep, idx)
    return out.reshape(batch, hist, dim)


# speedup vs baseline: 11.6639x; 1.0056x over previous
import jax
import jax.numpy as jnp
from jax.experimental import pallas as pl
from jax.experimental.pallas import tpu as pltpu
from jax.experimental.pallas import tpu_sc as plsc

WINDOW = 256
REP = 512


def kernel(tgt, emb_weight):
    batch, hist = tgt.shape
    n = batch * hist
    vocab, dim = emb_weight.shape
    table_rep = jnp.tile(emb_weight, (REP, 1))
    spread = (jnp.arange(n, dtype=jnp.int32) % REP) * vocab
    idx = (tgt.reshape(-1).astype(jnp.int32) + spread).reshape(1, n)

    mesh = plsc.VectorSubcoreMesh(core_axis_name="core",
                                  subcore_axis_name="subcore")

    @pl.kernel(out_type=jax.ShapeDtypeStruct((n, dim), emb_weight.dtype),
               mesh=mesh,
               scratch_types=[pltpu.SemaphoreType.DMA])
    def gather_kernel(table_hbm, idx_hbm, out_hbm, sem):
        def body(idx_vmem, out_vmem):
            pltpu.async_copy(table_hbm.at[idx_vmem.at[0]], out_vmem, sem).wait()

        pltpu.emit_pipeline(
            body,
            grid=(n // WINDOW,),
            in_specs=[pl.BlockSpec((1, WINDOW), index_map=lambda i: (0, i))],
            out_specs=[pl.BlockSpec((WINDOW, dim), index_map=lambda i: (i, 0))],
            core_axis_name=("core", "subcore"),
            dimension_semantics=(pltpu.PARALLEL,),
        )(idx_hbm, out_hbm)

    out = gather_kernel(table_rep, idx)
    return out.reshape(batch, hist, dim)


# speedup vs baseline: 20.5116x; 1.7586x over previous
import jax
import jax.numpy as jnp
from jax.experimental import pallas as pl
from jax.experimental.pallas import tpu as pltpu
from jax.experimental.pallas import tpu_sc as plsc

WINDOW = 256
REP = 64


def kernel(tgt, emb_weight):
    batch, hist = tgt.shape
    n = batch * hist
    vocab, dim = emb_weight.shape
    table_rep = jnp.tile(emb_weight, (REP, 1))
    spread = (jnp.arange(n, dtype=jnp.int32) % REP) * vocab
    idx = (tgt.reshape(-1).astype(jnp.int32) + spread).reshape(1, n)

    mesh = plsc.VectorSubcoreMesh(core_axis_name="core",
                                  subcore_axis_name="subcore")

    @pl.kernel(out_type=jax.ShapeDtypeStruct((n, dim), emb_weight.dtype),
               mesh=mesh,
               scratch_types=[pltpu.VMEM_SHARED((vocab * REP, dim), jnp.float32),
                              pltpu.SemaphoreType.DMA])
    def gather_kernel(table_hbm, idx_hbm, out_hbm, table_spmem, sem):
        @pl.when(jax.lax.axis_index("subcore") == 0)
        def _():
            pltpu.sync_copy(table_hbm, table_spmem)

        plsc.subcore_barrier()

        def body(idx_vmem, out_vmem):
            pltpu.async_copy(table_spmem.at[idx_vmem.at[0]], out_vmem,
                             sem).wait()

        pltpu.emit_pipeline(
            body,
            grid=(n // WINDOW,),
            in_specs=[pl.BlockSpec((1, WINDOW), index_map=lambda i: (0, i))],
            out_specs=[pl.BlockSpec((WINDOW, dim), index_map=lambda i: (i, 0))],
            core_axis_name=("core", "subcore"),
            dimension_semantics=(pltpu.PARALLEL,),
        )(idx_hbm, out_hbm)

    out = gather_kernel(table_rep, idx)
    return out.reshape(batch, hist, dim)
